# baseline (speedup 1.0000x reference)
import jax
import jax.numpy as jnp
from jax.experimental import pallas as pl


def _xw_body(x_ref, w_ref, o_ref):
    o_ref[...] = jnp.dot(x_ref[...], w_ref[...],
                         preferred_element_type=jnp.float32).astype(jnp.bfloat16)


def _project(X, W, block_m=2000):
    n, d_in = X.shape
    d_out = W.shape[1]
    block_m = min(block_m, n)
    grid = (pl.cdiv(n, block_m),)
    return pl.pallas_call(
        _xw_body,
        grid=grid,
        in_specs=[
            pl.BlockSpec((block_m, d_in), lambda i: (i, 0)),
            pl.BlockSpec((d_in, d_out), lambda i: (0, 0)),
        ],
        out_specs=pl.BlockSpec((block_m, d_out), lambda i: (i, 0)),
        out_shape=jax.ShapeDtypeStruct((n, d_out), jnp.bfloat16),
    )(X, W)


def _layer1_body(a_ref, g_ref, b_ref, wn_ref, abf_ref, o_ref):
    a16 = a_ref[...].astype(jnp.bfloat16)
    abf_ref[...] = a16
    h = jnp.dot(a16, g_ref[...], preferred_element_type=jnp.float32)
    h = jnp.maximum(h + b_ref[...], 0.0)
    o_ref[...] = jnp.dot(h, wn_ref[...],
                         preferred_element_type=jnp.float32).astype(jnp.bfloat16)


def _layer1(A, G, b, W_next, block_m=416):
    n = A.shape[0]
    d = G.shape[1]
    d_next = W_next.shape[1]
    grid = (pl.cdiv(n, block_m),)
    return pl.pallas_call(
        _layer1_body,
        grid=grid,
        in_specs=[
            pl.BlockSpec((block_m, n), lambda i: (i, 0)),
            pl.BlockSpec((n, d), lambda i: (0, 0)),
            pl.BlockSpec((1, d), lambda i: (0, 0)),
            pl.BlockSpec((d, d_next), lambda i: (0, 0)),
        ],
        out_specs=[
            pl.BlockSpec((block_m, n), lambda i: (i, 0)),
            pl.BlockSpec((block_m, d_next), lambda i: (i, 0)),
        ],
        out_shape=[
            jax.ShapeDtypeStruct((n, n), jnp.bfloat16),
            jax.ShapeDtypeStruct((n, d_next), jnp.bfloat16),
        ],
    )(A, G, b.reshape(1, d), W_next)


def _layer_body(a_ref, g_ref, b_ref, wn_ref, o_ref):
    h = jnp.dot(a_ref[...], g_ref[...], preferred_element_type=jnp.float32)
    h = jnp.maximum(h + b_ref[...], 0.0)
    o_ref[...] = jnp.dot(h, wn_ref[...],
                         preferred_element_type=jnp.float32).astype(jnp.bfloat16)


def _layer(A, G, b, W_next, block_m=1152):
    n = A.shape[0]
    d = G.shape[1]
    d_next = W_next.shape[1]
    grid = (pl.cdiv(n, block_m),)
    return pl.pallas_call(
        _layer_body,
        grid=grid,
        in_specs=[
            pl.BlockSpec((block_m, n), lambda i: (i, 0)),
            pl.BlockSpec((n, d), lambda i: (0, 0)),
            pl.BlockSpec((1, d), lambda i: (0, 0)),
            pl.BlockSpec((d, d_next), lambda i: (0, 0)),
        ],
        out_specs=pl.BlockSpec((block_m, d_next), lambda i: (i, 0)),
        out_shape=jax.ShapeDtypeStruct((n, d_next), jnp.bfloat16),
    )(A, G, b.reshape(1, d), W_next)


def _last_body(a_ref, g_ref, b_ref, o_ref):
    h = jnp.dot(a_ref[...], g_ref[...], preferred_element_type=jnp.float32)
    o_ref[...] = jnp.maximum(h + b_ref[...], 0.0)


def _last_layer(A, G, b, block_m=1152):
    n = A.shape[0]
    d = G.shape[1]
    grid = (pl.cdiv(n, block_m),)
    return pl.pallas_call(
        _last_body,
        grid=grid,
        in_specs=[
            pl.BlockSpec((block_m, n), lambda i: (i, 0)),
            pl.BlockSpec((n, d), lambda i: (0, 0)),
            pl.BlockSpec((1, d), lambda i: (0, 0)),
        ],
        out_specs=pl.BlockSpec((block_m, d), lambda i: (i, 0)),
        out_shape=jax.ShapeDtypeStruct((n, d), jnp.float32),
    )(A, G, b.reshape(1, d))


def kernel(X, A_hat, W1, b1, W2, b2, W3, b3, W4, b4):
    G = _project(X, W1)
    A16, G = _layer1(A_hat, G, b1, W2)
    G = _layer(A16, G, b2, W3)
    G = _layer(A16, G, b3, W4)
    return _last_layer(A16, G, b4)


# speedup vs baseline: 1.0052x; 1.0052x over previous
import jax
import jax.numpy as jnp
from jax.experimental import pallas as pl


def _xw_body(x_ref, w_ref, o_ref):
    o_ref[...] = jnp.dot(x_ref[...], w_ref[...],
                         preferred_element_type=jnp.float32).astype(jnp.bfloat16)


def _project(X, W, block_m=2000):
    n, d_in = X.shape
    d_out = W.shape[1]
    block_m = min(block_m, n)
    grid = (pl.cdiv(n, block_m),)
    return pl.pallas_call(
        _xw_body,
        grid=grid,
        in_specs=[
            pl.BlockSpec((block_m, d_in), lambda i: (i, 0)),
            pl.BlockSpec((d_in, d_out), lambda i: (0, 0)),
        ],
        out_specs=pl.BlockSpec((block_m, d_out), lambda i: (i, 0)),
        out_shape=jax.ShapeDtypeStruct((n, d_out), jnp.bfloat16),
    )(X, W)


def _layer1_body(a_ref, g_ref, b_ref, wn_ref, abf_ref, o_ref):
    a16 = a_ref[...].astype(jnp.bfloat16)
    abf_ref[...] = a16
    h = jnp.dot(a16, g_ref[...], preferred_element_type=jnp.float32)
    h = jnp.maximum(h + b_ref[...], 0.0)
    o_ref[...] = jnp.dot(h, wn_ref[...],
                         preferred_element_type=jnp.float32).astype(jnp.bfloat16)


def _layer1(A, G, b, W_next, block_m=416):
    n = A.shape[0]
    d = G.shape[1]
    d_next = W_next.shape[1]
    grid = (pl.cdiv(n, block_m),)
    return pl.pallas_call(
        _layer1_body,
        grid=grid,
        in_specs=[
            pl.BlockSpec((block_m, n), lambda i: (i, 0)),
            pl.BlockSpec((n, d), lambda i: (0, 0)),
            pl.BlockSpec((1, d), lambda i: (0, 0)),
            pl.BlockSpec((d, d_next), lambda i: (0, 0)),
        ],
        out_specs=[
            pl.BlockSpec((block_m, n), lambda i: (i, 0)),
            pl.BlockSpec((block_m, d_next), lambda i: (i, 0)),
        ],
        out_shape=[
            jax.ShapeDtypeStruct((n, n), jnp.bfloat16),
            jax.ShapeDtypeStruct((n, d_next), jnp.bfloat16),
        ],
    )(A, G, b.reshape(1, d), W_next)


def _layer_body(a_ref, g_ref, b_ref, wn_ref, o_ref):
    h = jnp.dot(a_ref[...], g_ref[...], preferred_element_type=jnp.float32)
    h = jnp.maximum(h + b_ref[...], 0.0)
    o_ref[...] = jnp.dot(h, wn_ref[...],
                         preferred_element_type=jnp.float32).astype(jnp.bfloat16)


def _layer(A, G, b, W_next, block_m=1024):
    n = A.shape[0]
    d = G.shape[1]
    d_next = W_next.shape[1]
    grid = (pl.cdiv(n, block_m),)
    return pl.pallas_call(
        _layer_body,
        grid=grid,
        in_specs=[
            pl.BlockSpec((block_m, n), lambda i: (i, 0)),
            pl.BlockSpec((n, d), lambda i: (0, 0)),
            pl.BlockSpec((1, d), lambda i: (0, 0)),
            pl.BlockSpec((d, d_next), lambda i: (0, 0)),
        ],
        out_specs=pl.BlockSpec((block_m, d_next), lambda i: (i, 0)),
        out_shape=jax.ShapeDtypeStruct((n, d_next), jnp.bfloat16),
    )(A, G, b.reshape(1, d), W_next)


def _last_body(a_ref, g_ref, b_ref, o_ref):
    h = jnp.dot(a_ref[...], g_ref[...], preferred_element_type=jnp.float32)
    o_ref[...] = jnp.maximum(h + b_ref[...], 0.0)


def _last_layer(A, G, b, block_m=1024):
    n = A.shape[0]
    d = G.shape[1]
    grid = (pl.cdiv(n, block_m),)
    return pl.pallas_call(
        _last_body,
        grid=grid,
        in_specs=[
            pl.BlockSpec((block_m, n), lambda i: (i, 0)),
            pl.BlockSpec((n, d), lambda i: (0, 0)),
            pl.BlockSpec((1, d), lambda i: (0, 0)),
        ],
        out_specs=pl.BlockSpec((block_m, d), lambda i: (i, 0)),
        out_shape=jax.ShapeDtypeStruct((n, d), jnp.float32),
    )(A, G, b.reshape(1, d))


def kernel(X, A_hat, W1, b1, W2, b2, W3, b3, W4, b4):
    G = _project(X, W1)
    A16, G = _layer1(A_hat, G, b1, W2)
    G = _layer(A16, G, b2, W3)
    G = _layer(A16, G, b3, W4)
    return _last_layer(A16, G, b4)
